# Initial kernel scaffold; baseline (speedup 1.0000x reference)
import jax
import jax.numpy as jnp
from jax.experimental import pallas as pl


def kernel(ft_tf, ft_gene, adj_tf_tf, adj_tf_gene, adj_gene_tf, adj_gene_gene, params):
    raise NotImplementedError("write your pallas kernel here")


# speedup vs baseline: 1.1005x; 1.1005x over previous
import functools

import jax
import jax.numpy as jnp
from jax.experimental import pallas as pl
from jax.experimental.pallas import tpu as pltpu

B = 16
TF_N = 384
GENE_N = 3072
ALL_N = TF_N + GENE_N
D1 = 128
D2 = 64
POOL = 8
BLK = 384
HEAD_KBLK = 1024


def _elu(x):
    return jnp.where(x > 0, x, jnp.exp(jnp.minimum(x, 0.0)) - 1.0)


def _layer1_body(xt_self_ref, adj_t_ref, adj_g_ref, xt_tf_ref, xt_gene_ref,
                 w13_ref, b1_ref, g1_ref, be1_ref, w2cat_ref,
                 ya_ref, yg_ref, s_ref):
    u1 = jnp.dot(adj_t_ref[...], xt_tf_ref[...],
                 preferred_element_type=jnp.float32)
    u2 = jnp.dot(adj_g_ref[...], xt_gene_ref[...],
                 preferred_element_type=jnp.float32)
    x0 = xt_self_ref[...]
    w13 = w13_ref[...]
    h = (x0[:, :, None] * w13[0][None, None, :]
         + u1[:, :, None] * w13[1][None, None, :]
         + u2[:, :, None] * w13[2][None, None, :]) * (1.0 / 3.0)
    h = h + b1_ref[...][0][None, None, :]
    mu = jnp.mean(h, axis=-1, keepdims=True)
    var = jnp.mean((h - mu) ** 2, axis=-1, keepdims=True)
    x1 = (h - mu) * jax.lax.rsqrt(var + 1e-5)
    x1 = x1 * g1_ref[...][None, None, :] + be1_ref[...][None, None, :]
    x1 = _elu(x1)
    m = x1.shape[0]
    proj = jnp.dot(x1.reshape(m * B, D1), w2cat_ref[...],
                   preferred_element_type=jnp.float32)
    ya_ref[...] = proj[:, 0:D2].reshape(m, B, D2)
    yg_ref[...] = proj[:, D2:2 * D2].reshape(m, B, D2)
    s_ref[...] = proj[:, 2 * D2:3 * D2].reshape(m, B, D2)


def _layer1_call(xt_self, adj_t, adj_g, xt_tf, xt_gene,
                 w13, b1, g1, be1, w2cat, grid_m):
    m_total = xt_self.shape[0]
    n_blk = m_total // grid_m
    out_sds = [jax.ShapeDtypeStruct((m_total, B, D2), jnp.float32)] * 3
    full2 = lambda a: pl.BlockSpec(a.shape, lambda i: (0,) * a.ndim)
    in_specs = [
        pl.BlockSpec((grid_m, B), lambda i: (i, 0)),
        pl.BlockSpec((grid_m, TF_N), lambda i: (i, 0)),
        pl.BlockSpec((grid_m, GENE_N), lambda i: (i, 0)),
        full2(xt_tf), full2(xt_gene),
        full2(w13), full2(b1), full2(g1), full2(be1), full2(w2cat),
    ]
    out_specs = [pl.BlockSpec((grid_m, B, D2), lambda i: (i, 0, 0))] * 3
    return pl.pallas_call(
        _layer1_body,
        grid=(n_blk,),
        in_specs=in_specs,
        out_specs=out_specs,
        out_shape=out_sds,
    )(xt_self, adj_t, adj_g, xt_tf, xt_gene, w13, b1, g1, be1, w2cat)


def _layer2_body(adj_t_ref, adj_g_ref, ya_ref, yb_ref, s_ref,
                 b2t_ref, g2t_ref, be2t_ref, gsum_ref, gbc_ref, out_ref):
    agg = jnp.dot(adj_t_ref[...], ya_ref[...],
                  preferred_element_type=jnp.float32)
    agg = agg + jnp.dot(adj_g_ref[...], yb_ref[...],
                        preferred_element_type=jnp.float32)
    h = (agg + s_ref[...]) * (1.0 / 3.0) + b2t_ref[...][0][None, :]
    gsum = gsum_ref[...]
    gbc = gbc_ref[...]
    mu = jnp.dot(h, gsum, preferred_element_type=jnp.float32) * (1.0 / D2)
    mub = jnp.dot(mu, gbc, preferred_element_type=jnp.float32)
    hc = h - mub
    var = jnp.dot(hc * hc, gsum, preferred_element_type=jnp.float32) * (1.0 / D2)
    varb = jnp.dot(var, gbc, preferred_element_type=jnp.float32)
    x2 = hc * jax.lax.rsqrt(varb + 1e-5)
    x2 = x2 * g2t_ref[...][0][None, :] + be2t_ref[...][0][None, :]
    x2 = _elu(x2)
    m = x2.shape[0]
    out_ref[...] = jnp.max(x2.reshape(m // POOL, POOL, B * D2), axis=1)


def _layer2_call(adj_t, adj_g, ya, yb, s, b2t, g2t, be2t, gsum, gbc, grid_m):
    m_total = adj_t.shape[0]
    n_blk = m_total // grid_m
    full2 = lambda a: pl.BlockSpec(a.shape, lambda i: (0,) * a.ndim)
    in_specs = [
        pl.BlockSpec((grid_m, TF_N), lambda i: (i, 0)),
        pl.BlockSpec((grid_m, GENE_N), lambda i: (i, 0)),
        full2(ya), full2(yb),
        pl.BlockSpec((grid_m, B * D2), lambda i: (i, 0)),
        full2(b2t), full2(g2t), full2(be2t), full2(gsum), full2(gbc),
    ]
    out_specs = pl.BlockSpec((grid_m // POOL, B * D2), lambda i: (i, 0))
    return pl.pallas_call(
        _layer2_body,
        grid=(n_blk,),
        in_specs=in_specs,
        out_specs=out_specs,
        out_shape=jax.ShapeDtypeStruct((m_total // POOL, B * D2), jnp.float32),
    )(adj_t, adj_g, ya, yb, s, b2t, g2t, be2t, gsum, gbc)


def _head_body(flat_ref, gew_ref, geb_ref, recw_ref, recb_ref, x0_ref,
               fc1w_ref, fc1b_ref, fc2w_ref, fc2b_ref, clsw_ref, clsb_ref,
               logits_ref, dec_ref, cell_ref, acc_ref):
    k = pl.program_id(0)
    part = jnp.dot(flat_ref[...], gew_ref[...],
                   preferred_element_type=jnp.float32)

    @pl.when(k == 0)
    def _():
        acc_ref[...] = part

    @pl.when(k > 0)
    def _():
        acc_ref[...] = acc_ref[...] + part

    @pl.when(k == pl.num_programs(0) - 1)
    def _():
        xh = jnp.maximum(acc_ref[...] + geb_ref[...][None, :], 0.0)
        dec_ref[...] = (jnp.dot(xh, recw_ref[...],
                                preferred_element_type=jnp.float32)
                        + recb_ref[...][None, :])
        xnn = jnp.maximum(jnp.dot(x0_ref[...], fc1w_ref[...],
                                  preferred_element_type=jnp.float32)
                          + fc1b_ref[...][None, :], 0.0)
        xnn = jnp.maximum(jnp.dot(xnn, fc2w_ref[...],
                                  preferred_element_type=jnp.float32)
                          + fc2b_ref[...][None, :], 0.0)
        cell = jnp.concatenate([xh, xnn], axis=1)
        cell_ref[...] = cell
        logits_ref[...] = (jnp.dot(cell, clsw_ref[...],
                                   preferred_element_type=jnp.float32)
                           + clsb_ref[...][None, :])


def _head_call(flat, gew, geb, recw, recb, x0, fc1w, fc1b, fc2w, fc2b,
               clsw, clsb):
    n_k = flat.shape[1] // HEAD_KBLK
    full = lambda a: pl.BlockSpec(a.shape, lambda i: (0,) * a.ndim)
    in_specs = [
        pl.BlockSpec((B, HEAD_KBLK), lambda i: (0, i)),
        pl.BlockSpec((HEAD_KBLK, D1), lambda i: (i, 0)),
        full(geb), full(recw), full(recb), full(x0),
        full(fc1w), full(fc1b), full(fc2w), full(fc2b),
        full(clsw), full(clsb),
    ]
    out_sds = [
        jax.ShapeDtypeStruct((B, 10), jnp.float32),
        jax.ShapeDtypeStruct((B, ALL_N), jnp.float32),
        jax.ShapeDtypeStruct((B, 2 * D1), jnp.float32),
    ]
    out_specs = [
        pl.BlockSpec((B, 10), lambda i: (0, 0)),
        pl.BlockSpec((B, ALL_N), lambda i: (0, 0)),
        pl.BlockSpec((B, 2 * D1), lambda i: (0, 0)),
    ]
    return pl.pallas_call(
        _head_body,
        grid=(n_k,),
        in_specs=in_specs,
        out_specs=out_specs,
        out_shape=out_sds,
        scratch_shapes=[pltpu.VMEM((B, D1), jnp.float32)],
    )(flat, gew, geb, recw, recb, x0, fc1w, fc1b, fc2w, fc2b, clsw, clsb)


@functools.partial(jax.jit, static_argnums=())
def kernel(ft_tf, ft_gene, adj_tf_tf, adj_tf_gene, adj_gene_tf,
           adj_gene_gene, params):
    p1, p2 = params['hgc1'], params['hgc2']
    xt_tf = ft_tf[:, :, 0].T
    xt_gene = ft_gene[:, :, 0].T

    def w13_of(k):
        q = p1[k]
        return jnp.concatenate([q['w_self'], q['w_rel_tf'], q['w_rel_gene']],
                               axis=0)

    def w2cat_of(k):
        return jnp.concatenate([p2['tf']['w_rel_' + k],
                                p2['gene']['w_rel_' + k],
                                p2[k]['w_self']], axis=1)

    g1 = params['ln1_g']
    be1 = params['ln1_b']
    ya_tf, yg_tf, s_tf = _layer1_call(
        xt_tf, adj_tf_tf, adj_tf_gene, xt_tf, xt_gene,
        w13_of('tf'), p1['tf']['bias'], g1, be1, w2cat_of('tf'), TF_N)
    ya_ge, yg_ge, s_ge = _layer1_call(
        xt_gene, adj_gene_tf, adj_gene_gene, xt_tf, xt_gene,
        w13_of('gene'), p1['gene']['bias'], g1, be1, w2cat_of('gene'), BLK)

    r2 = lambda a: a.reshape(a.shape[0], B * D2)
    tile16 = lambda v: jnp.tile(v.reshape(-1), (B,))
    gsum = jnp.repeat(jnp.eye(B, dtype=jnp.float32), D2, axis=0)
    gbc = gsum.T
    g2t = tile16(params['ln2_g']).reshape(1, B * D2)
    be2t = tile16(params['ln2_b']).reshape(1, B * D2)

    pooled_tf = _layer2_call(
        adj_tf_tf, adj_tf_gene, r2(ya_tf), r2(ya_ge), r2(s_tf),
        tile16(p2['tf']['bias']).reshape(1, B * D2), g2t, be2t,
        gsum, gbc, TF_N)
    pooled_ge = _layer2_call(
        adj_gene_tf, adj_gene_gene, r2(yg_tf), r2(yg_ge), r2(s_ge),
        tile16(p2['gene']['bias']).reshape(1, B * D2), g2t, be2t,
        gsum, gbc, BLK)

    pooled = jnp.concatenate([pooled_tf, pooled_ge], axis=0)
    flat = pooled.reshape(ALL_N // POOL, B, D2).transpose(1, 0, 2)
    flat = flat.reshape(B, (ALL_N // POOL) * D2)

    x0 = jnp.concatenate([xt_tf.T, xt_gene.T], axis=1)
    logits, x_decode, cell = _head_call(
        flat, params['ge_W'], params['ge_b'], params['rec_W'],
        params['rec_b'], x0, params['fc1_W'], params['fc1_b'],
        params['fc2_W'], params['fc2_b'], params['cls_W'], params['cls_b'])
    return (logits, x_decode, cell)
